# baseline (speedup 1.0000x reference)
import functools

import jax
import jax.numpy as jnp
from jax import lax
from jax.experimental import pallas as pl
from jax.experimental.pallas import tpu as pltpu
from jax.experimental.pallas import tpu_sc as plsc

NUM_EMB = 1_000_000
D_IN = 64
D_OUT = 32
SCALE = float(D_OUT) ** 0.5

NC = 2
NS = 16
NW = NC * NS

B_TOTAL = 4096 * 200
B_PER_W = B_TOTAL // NW
CHUNK = 128
N_CHUNKS = B_PER_W // CHUNK
K = 8
N_GROUPS = N_CHUNKS // K
GROUP_ROWS = K * CHUNK

ROWS_BLOCK = 8192
N_BLOCKS = pl.cdiv(NUM_EMB, ROWS_BLOCK)
PACK_LINES = ROWS_BLOCK // 4
PTAB_ROWS = N_BLOCKS * ROWS_BLOCK


def _project_body(tabT_ref, w_ref, out_ref):
    p = lax.dot_general(w_ref[...], tabT_ref[...],
                        (((1,), (0,)), ((), ())),
                        preferred_element_type=jnp.float32)
    t = lax.transpose(p * SCALE, (1, 0))
    for j in range(4):
        out_ref[:, 32 * j:32 * (j + 1)] = t[2048 * j:2048 * (j + 1), :]


def _project(tabT, w):
    return pl.pallas_call(
        _project_body,
        grid=(N_BLOCKS,),
        in_specs=[
            pl.BlockSpec((D_IN, ROWS_BLOCK), lambda i: (0, i)),
            pl.BlockSpec((D_OUT, D_IN), lambda i: (0, 0)),
        ],
        out_specs=pl.BlockSpec((PACK_LINES, 128), lambda i: (i, 0)),
        out_shape=jax.ShapeDtypeStruct((N_BLOCKS * PACK_LINES, 128), jnp.float32),
    )(tabT, w)


_MESH = plsc.VectorSubcoreMesh(core_axis_name="c", subcore_axis_name="s",
                               num_cores=NC, num_subcores=NS)


@functools.partial(
    pl.kernel,
    out_type=jax.ShapeDtypeStruct((B_TOTAL, D_OUT), jnp.float32),
    mesh=_MESH,
    compiler_params=pltpu.CompilerParams(use_tc_tiling_on_sc=False),
    scratch_types=[
        pltpu.VMEM((B_PER_W,), jnp.int32),
        pltpu.VMEM((GROUP_ROWS, D_OUT), jnp.float32),
        pltpu.SemaphoreType.DMA,
    ],
)
def _gather(idx_hbm, ptab_hbm, out_hbm, idx_v, rows_v, gsem):
    wid = lax.axis_index("s") * NC + lax.axis_index("c")
    pltpu.sync_copy(idx_hbm.at[pl.ds(wid * B_PER_W, B_PER_W)], idx_v)

    def xform(k, carry):
        v = idx_v[pl.ds(k * 16, 16)]
        g = ((v & -8192) + ((v & 2047) << 2)) + ((v >> 11) & 3)
        idx_v[pl.ds(k * 16, 16)] = g
        return carry

    lax.fori_loop(0, B_PER_W // 16, xform, 0)

    out_base = wid * B_PER_W

    def group(g, carry):
        for b in range(K):
            pltpu.async_copy(
                ptab_hbm.at[idx_v.at[pl.ds((g * K + b) * CHUNK, CHUNK)]],
                rows_v.at[pl.ds(b * CHUNK, CHUNK)],
                gsem,
            )
        for b in range(K):
            pltpu.make_async_copy(
                ptab_hbm.at[idx_v.at[pl.ds((g * K + b) * CHUNK, CHUNK)]],
                rows_v.at[pl.ds(b * CHUNK, CHUNK)],
                gsem,
            ).wait()
        pltpu.sync_copy(rows_v,
                        out_hbm.at[pl.ds(out_base + g * GROUP_ROWS, GROUP_ROWS)])
        return carry

    lax.fori_loop(0, N_GROUPS, group, 0)


def kernel(x, table, W):
    ptab = _project(table.T, W)
    idxT = x.T.reshape(B_TOTAL).astype(jnp.int32)
    outT = _gather(idxT, ptab.reshape(PTAB_ROWS, D_OUT))
    return jnp.swapaxes(outT.reshape(200, 4096, D_OUT), 0, 1)


# speedup vs baseline: 1.4349x; 1.4349x over previous
import functools

import jax
import jax.numpy as jnp
from jax import lax
from jax.experimental import pallas as pl
from jax.experimental.pallas import tpu as pltpu
from jax.experimental.pallas import tpu_sc as plsc

NUM_EMB = 1_000_000
D_IN = 64
D_OUT = 32
SCALE = float(D_OUT) ** 0.5

NC = 2
NS = 16
NW = NC * NS

B_TOTAL = 4096 * 200
B_PER_W = B_TOTAL // NW
CHUNK = 128
N_CHUNKS = B_PER_W // CHUNK
K = 8
N_GROUPS = N_CHUNKS // K
GROUP_ROWS = K * CHUNK

ROWS_BLOCK = 8192
N_BLOCKS = pl.cdiv(NUM_EMB, ROWS_BLOCK)
PACK_LINES = ROWS_BLOCK // 4
PTAB_ROWS = N_BLOCKS * ROWS_BLOCK


def _project_body(tabT_ref, w_ref, out_ref):
    p = lax.dot_general(w_ref[...], tabT_ref[...],
                        (((1,), (0,)), ((), ())),
                        preferred_element_type=jnp.float32)
    t = lax.transpose(p * SCALE, (1, 0))
    for j in range(4):
        out_ref[:, 32 * j:32 * (j + 1)] = t[2048 * j:2048 * (j + 1), :]


def _project(tabT, w):
    return pl.pallas_call(
        _project_body,
        grid=(N_BLOCKS,),
        in_specs=[
            pl.BlockSpec((D_IN, ROWS_BLOCK), lambda i: (0, i)),
            pl.BlockSpec((D_OUT, D_IN), lambda i: (0, 0)),
        ],
        out_specs=pl.BlockSpec((PACK_LINES, 128), lambda i: (i, 0)),
        out_shape=jax.ShapeDtypeStruct((N_BLOCKS * PACK_LINES, 128), jnp.float32),
    )(tabT, w)


_MESH = plsc.VectorSubcoreMesh(core_axis_name="c", subcore_axis_name="s",
                               num_cores=NC, num_subcores=NS)


N_UNITS = 50
UNIT_ROWS = 512
STAGE = 8 * 4096


@functools.partial(
    pl.kernel,
    out_type=jax.ShapeDtypeStruct((B_TOTAL, D_OUT), jnp.float32),
    mesh=_MESH,
    compiler_params=pltpu.CompilerParams(use_tc_tiling_on_sc=False,
                                         needs_layout_passes=False),
    scratch_types=[
        pltpu.VMEM((STAGE,), jnp.int32),
        pltpu.VMEM((2 * UNIT_ROWS,), jnp.int32),
        pltpu.VMEM((2 * UNIT_ROWS, D_OUT), jnp.float32),
        pltpu.SemaphoreType.DMA,
        pltpu.SemaphoreType.DMA,
    ],
)
def _gather(idx_hbm, ptab_hbm, out_hbm, idx_v, gidx_v, rows_v, gsem, wsem):
    wid = lax.axis_index("s") * NC + lax.axis_index("c")
    start = jnp.minimum(((wid * N_UNITS) // 8) * 4096, B_TOTAL - STAGE)
    pltpu.sync_copy(idx_hbm.at[pl.ds(start, STAGE)], idx_v)

    lane = lax.iota(jnp.int32, 16)

    def build_and_fire(m, slot):
        mg = wid * N_UNITS + m
        l = mg // 8
        j = mg % 8
        base = l * 4096 + 128 * j - start
        for mv in range(UNIT_ROWS // 16):
            k = 16 * mv + lane
            sp = base + 1024 * (k & 3) + (k >> 2)
            v = plsc.load_gather(idx_v, [sp])
            gv = ((v & -8192) + ((v & 2047) << 2)) + ((v >> 11) & 3)
            gidx_v[pl.ds(slot * UNIT_ROWS + 16 * mv, 16)] = gv
        for q in range(4):
            pltpu.async_copy(
                ptab_hbm.at[gidx_v.at[pl.ds(slot * UNIT_ROWS + 128 * q, CHUNK)]],
                rows_v.at[pl.ds(slot * UNIT_ROWS + 128 * q, CHUNK)],
                gsem,
            )

    def drain_gathers(slot):
        for q in range(4):
            pltpu.make_async_copy(
                ptab_hbm.at[gidx_v.at[pl.ds(slot * UNIT_ROWS + 128 * q, CHUNK)]],
                rows_v.at[pl.ds(slot * UNIT_ROWS + 128 * q, CHUNK)],
                gsem,
            ).wait()

    def write_copy(m, slot):
        mg = wid * N_UNITS + m
        dst = 4096 * (mg // 8) + 512 * (mg % 8)
        return pltpu.make_async_copy(
            rows_v.at[pl.ds(slot * UNIT_ROWS, UNIT_ROWS)],
            out_hbm.at[pl.ds(dst, UNIT_ROWS)],
            wsem,
        )

    build_and_fire(0, 0)

    def unit(m, carry):
        slot = m % 2
        nslot = 1 - slot

        @pl.when(m >= 1)
        def _():
            write_copy(m - 1, nslot).wait()

        @pl.when(m + 1 < N_UNITS)
        def _():
            build_and_fire(m + 1, nslot)

        drain_gathers(slot)
        write_copy(m, slot).start()
        return carry

    lax.fori_loop(0, N_UNITS, unit, 0)
    write_copy(N_UNITS - 1, (N_UNITS - 1) % 2).wait()


def _unpack_body(v_ref, out_ref):
    vT = lax.transpose(v_ref[...], (1, 0))
    for u in range(4):
        out_ref[0, :, 1024 * u:1024 * (u + 1)] = vT[32 * u:32 * (u + 1), :]


def _unpack(v2):
    return pl.pallas_call(
        _unpack_body,
        grid=(200,),
        in_specs=[pl.BlockSpec((1024, 128), lambda l: (l, 0))],
        out_specs=pl.BlockSpec((1, D_OUT, 4096), lambda l: (l, 0, 0)),
        out_shape=jax.ShapeDtypeStruct((200, D_OUT, 4096), jnp.float32),
    )(v2)


def kernel(x, table, W):
    ptab = _project(table.T, W)
    idxT = x.T.reshape(B_TOTAL).astype(jnp.int32)
    v2 = _gather(idxT, ptab.reshape(PTAB_ROWS, D_OUT))
    out3 = _unpack(v2.reshape(B_TOTAL // 4, 128))
    return jnp.transpose(out3, (2, 0, 1))


# speedup vs baseline: 1.7216x; 1.1998x over previous
import functools

import jax
import jax.numpy as jnp
from jax import lax
from jax.experimental import pallas as pl
from jax.experimental.pallas import tpu as pltpu
from jax.experimental.pallas import tpu_sc as plsc

NUM_EMB = 1_000_000
D_IN = 64
D_OUT = 32
SCALE = float(D_OUT) ** 0.5

NC = 2
NS = 16
NW = NC * NS

B_TOTAL = 4096 * 200
B_PER_W = B_TOTAL // NW
CHUNK = 128
N_CHUNKS = B_PER_W // CHUNK
K = 8
N_GROUPS = N_CHUNKS // K
GROUP_ROWS = K * CHUNK

ROWS_BLOCK = 16384
N_BLOCKS = pl.cdiv(NUM_EMB, ROWS_BLOCK)
PACK_LINES = ROWS_BLOCK // 4
PTAB_ROWS = N_BLOCKS * ROWS_BLOCK


def _project_body(tabT_ref, w_ref, out_ref):
    p = lax.dot_general(w_ref[...], tabT_ref[...],
                        (((1,), (0,)), ((), ())),
                        preferred_element_type=jnp.float32)
    t = lax.transpose(p * SCALE, (1, 0))
    for j in range(4):
        out_ref[:, 32 * j:32 * (j + 1)] = t[PACK_LINES * j:PACK_LINES * (j + 1), :]


def _project(tabT, w):
    return pl.pallas_call(
        _project_body,
        grid=(N_BLOCKS,),
        in_specs=[
            pl.BlockSpec((D_IN, ROWS_BLOCK), lambda i: (0, i)),
            pl.BlockSpec((D_OUT, D_IN), lambda i: (0, 0)),
        ],
        out_specs=pl.BlockSpec((PACK_LINES, 128), lambda i: (i, 0)),
        out_shape=jax.ShapeDtypeStruct((N_BLOCKS * PACK_LINES, 128), jnp.float32),
    )(tabT, w)


_MESH = plsc.VectorSubcoreMesh(core_axis_name="c", subcore_axis_name="s",
                               num_cores=NC, num_subcores=NS)


N_UNITS = 50
UNIT_ROWS = 512
STAGE = 8 * 4096


@functools.partial(
    pl.kernel,
    out_type=jax.ShapeDtypeStruct((B_TOTAL, D_OUT), jnp.float32),
    mesh=_MESH,
    compiler_params=pltpu.CompilerParams(use_tc_tiling_on_sc=False,
                                         needs_layout_passes=False),
    scratch_types=[
        pltpu.VMEM((STAGE,), jnp.int32),
        pltpu.VMEM((2 * UNIT_ROWS,), jnp.int32),
        pltpu.VMEM((2 * UNIT_ROWS, D_OUT), jnp.float32),
        pltpu.SemaphoreType.DMA,
        pltpu.SemaphoreType.DMA,
    ],
)
def _gather(idx_hbm, ptab_hbm, out_hbm, idx_v, gidx_v, rows_v, gsem, wsem):
    wid = lax.axis_index("s") * NC + lax.axis_index("c")
    start = jnp.minimum(((wid * N_UNITS) // 8) * 4096, B_TOTAL - STAGE)
    pltpu.sync_copy(idx_hbm.at[pl.ds(start, STAGE)], idx_v)

    lane = lax.iota(jnp.int32, 16)

    def build_and_fire(m, slot):
        mg = wid * N_UNITS + m
        l = mg // 8
        j = mg % 8
        base = l * 4096 + 128 * j - start
        for mv in range(UNIT_ROWS // 16):
            k = 16 * mv + lane
            sp = base + 1024 * (k & 3) + (k >> 2)
            v = plsc.load_gather(idx_v, [sp])
            gv = ((v & -16384) + ((v & 4095) << 2)) + ((v >> 12) & 3)
            gidx_v[pl.ds(slot * UNIT_ROWS + 16 * mv, 16)] = gv
        for q in range(4):
            pltpu.async_copy(
                ptab_hbm.at[gidx_v.at[pl.ds(slot * UNIT_ROWS + 128 * q, CHUNK)]],
                rows_v.at[pl.ds(slot * UNIT_ROWS + 128 * q, CHUNK)],
                gsem,
            )

    def drain_gathers(slot):
        for q in range(4):
            pltpu.make_async_copy(
                ptab_hbm.at[gidx_v.at[pl.ds(slot * UNIT_ROWS + 128 * q, CHUNK)]],
                rows_v.at[pl.ds(slot * UNIT_ROWS + 128 * q, CHUNK)],
                gsem,
            ).wait()

    def write_copy(m, slot):
        mg = wid * N_UNITS + m
        dst = 4096 * (mg // 8) + 512 * (mg % 8)
        return pltpu.make_async_copy(
            rows_v.at[pl.ds(slot * UNIT_ROWS, UNIT_ROWS)],
            out_hbm.at[pl.ds(dst, UNIT_ROWS)],
            wsem,
        )

    build_and_fire(0, 0)

    def unit(m, carry):
        slot = m % 2
        nslot = 1 - slot

        @pl.when(m >= 1)
        def _():
            write_copy(m - 1, nslot).wait()

        @pl.when(m + 1 < N_UNITS)
        def _():
            build_and_fire(m + 1, nslot)

        drain_gathers(slot)
        write_copy(m, slot).start()
        return carry

    lax.fori_loop(0, N_UNITS, unit, 0)
    write_copy(N_UNITS - 1, (N_UNITS - 1) % 2).wait()


UNPACK_L = 4


def _unpack_body(v_ref, out_ref):
    vT = lax.transpose(v_ref[...], (1, 0))
    for li in range(UNPACK_L):
        for u in range(4):
            out_ref[li, :, 1024 * u:1024 * (u + 1)] = (
                vT[32 * u:32 * (u + 1), 1024 * li:1024 * (li + 1)])


def _unpack(v2):
    return pl.pallas_call(
        _unpack_body,
        grid=(200 // UNPACK_L,),
        in_specs=[pl.BlockSpec((1024 * UNPACK_L, 128), lambda l: (l, 0))],
        out_specs=pl.BlockSpec((UNPACK_L, D_OUT, 4096), lambda l: (l, 0, 0)),
        out_shape=jax.ShapeDtypeStruct((200, D_OUT, 4096), jnp.float32),
    )(v2)


def kernel(x, table, W):
    ptab = _project(table.T, W)
    idxT = x.T.reshape(B_TOTAL).astype(jnp.int32)
    v2 = _gather(idxT, ptab.reshape(PTAB_ROWS, D_OUT))
    out3 = _unpack(v2.reshape(B_TOTAL // 4, 128))
    return jnp.transpose(out3, (2, 0, 1))


# speedup vs baseline: 2.8805x; 1.6732x over previous
import functools

import jax
import jax.numpy as jnp
from jax import lax
from jax.experimental import pallas as pl
from jax.experimental.pallas import tpu as pltpu
from jax.experimental.pallas import tpu_sc as plsc

NUM_EMB = 1_000_000
D_IN = 64
D_OUT = 32
SCALE = float(D_OUT) ** 0.5

NC = 2
NS = 16
NW = NC * NS

B_TOTAL = 4096 * 200
B_PER_W = B_TOTAL // NW
CHUNK = 128
N_CHUNKS = B_PER_W // CHUNK
K = 8
N_GROUPS = N_CHUNKS // K
GROUP_ROWS = K * CHUNK

ROWS_BLOCK = 16384
N_BLOCKS = pl.cdiv(NUM_EMB, ROWS_BLOCK)
PACK_LINES = ROWS_BLOCK // 8
PTAB_ROWS = N_BLOCKS * ROWS_BLOCK
ROW_W = 16


def _project_body(tabT_ref, w_ref, out_ref):
    p = lax.dot_general(w_ref[...], tabT_ref[...],
                        (((1,), (0,)), ((), ())),
                        preferred_element_type=jnp.float32)
    c = pltpu.bitcast((p * SCALE).astype(jnp.bfloat16), jnp.float32)
    c8 = jnp.concatenate(
        [c[:, s * PACK_LINES:(s + 1) * PACK_LINES] for s in range(8)],
        axis=0)
    out_ref[...] = lax.transpose(c8, (1, 0))


def _project(tabT, w):
    return pl.pallas_call(
        _project_body,
        grid=(N_BLOCKS,),
        in_specs=[
            pl.BlockSpec((D_IN, ROWS_BLOCK), lambda i: (0, i)),
            pl.BlockSpec((D_OUT, D_IN), lambda i: (0, 0)),
        ],
        out_specs=pl.BlockSpec((PACK_LINES, 128), lambda i: (i, 0)),
        out_shape=jax.ShapeDtypeStruct((N_BLOCKS * PACK_LINES, 128), jnp.float32),
    )(tabT, w)


_MESH = plsc.VectorSubcoreMesh(core_axis_name="c", subcore_axis_name="s",
                               num_cores=NC, num_subcores=NS)


N_UNITS = 25
UNIT_ROWS = 1024
STAGE = 8 * 4096


@functools.partial(
    pl.kernel,
    out_type=jax.ShapeDtypeStruct((B_TOTAL, ROW_W), jnp.float32),
    mesh=_MESH,
    compiler_params=pltpu.CompilerParams(use_tc_tiling_on_sc=False,
                                         needs_layout_passes=False),
    scratch_types=[
        pltpu.VMEM((STAGE,), jnp.int32),
        pltpu.VMEM((2 * UNIT_ROWS,), jnp.int32),
        pltpu.VMEM((2 * UNIT_ROWS, ROW_W), jnp.float32),
        pltpu.SemaphoreType.DMA,
        pltpu.SemaphoreType.DMA,
    ],
)
def _gather(idx_hbm, ptab_hbm, out_hbm, idx_v, gidx_v, rows_v, gsem, wsem):
    wid = lax.axis_index("s") * NC + lax.axis_index("c")
    start = jnp.minimum(((wid * N_UNITS) // 4) * 4096, B_TOTAL - STAGE)
    pltpu.sync_copy(idx_hbm.at[pl.ds(start, STAGE)], idx_v)

    lane = lax.iota(jnp.int32, 16)

    def build_and_fire(m, slot):
        mg = wid * N_UNITS + m
        l = mg // 4
        j = mg % 4
        base = l * 4096 + 128 * j - start
        for mv in range(UNIT_ROWS // 16):
            k = 16 * mv + lane
            sp = base + 512 * (k & 7) + (k >> 3)
            v = plsc.load_gather(idx_v, [sp])
            gv = ((v & -16384) + ((v & 2047) << 3)) + ((v >> 11) & 7)
            gidx_v[pl.ds(slot * UNIT_ROWS + 16 * mv, 16)] = gv
        for q in range(8):
            pltpu.async_copy(
                ptab_hbm.at[gidx_v.at[pl.ds(slot * UNIT_ROWS + 128 * q, CHUNK)]],
                rows_v.at[pl.ds(slot * UNIT_ROWS + 128 * q, CHUNK)],
                gsem,
            )

    def drain_gathers(slot):
        for q in range(8):
            pltpu.make_async_copy(
                ptab_hbm.at[gidx_v.at[pl.ds(slot * UNIT_ROWS + 128 * q, CHUNK)]],
                rows_v.at[pl.ds(slot * UNIT_ROWS + 128 * q, CHUNK)],
                gsem,
            ).wait()

    def write_copy(m, slot):
        mg = wid * N_UNITS + m
        dst = 4096 * (mg // 4) + 1024 * (mg % 4)
        return pltpu.make_async_copy(
            rows_v.at[pl.ds(slot * UNIT_ROWS, UNIT_ROWS)],
            out_hbm.at[pl.ds(dst, UNIT_ROWS)],
            wsem,
        )

    build_and_fire(0, 0)

    def unit(m, carry):
        slot = m % 2
        nslot = 1 - slot

        @pl.when(m >= 1)
        def _():
            write_copy(m - 1, nslot).wait()

        @pl.when(m + 1 < N_UNITS)
        def _():
            build_and_fire(m + 1, nslot)

        drain_gathers(slot)
        write_copy(m, slot).start()
        return carry

    lax.fori_loop(0, N_UNITS, unit, 0)
    write_copy(N_UNITS - 1, (N_UNITS - 1) % 2).wait()


UNPACK_L = 4


def _unpack_body(v_ref, out_ref):
    vT = lax.transpose(v_ref[...], (1, 0))
    for li in range(UNPACK_L):
        for u in range(8):
            cs = vT[16 * u:16 * (u + 1), 512 * li:512 * (li + 1)]
            bf = pltpu.bitcast(cs, jnp.bfloat16)
            out_ref[li, :, 512 * u:512 * (u + 1)] = bf.astype(jnp.float32)


def _unpack(v2):
    return pl.pallas_call(
        _unpack_body,
        grid=(200 // UNPACK_L,),
        in_specs=[pl.BlockSpec((512 * UNPACK_L, 128), lambda l: (l, 0))],
        out_specs=pl.BlockSpec((UNPACK_L, D_OUT, 4096), lambda l: (l, 0, 0)),
        out_shape=jax.ShapeDtypeStruct((200, D_OUT, 4096), jnp.float32),
    )(v2)


def kernel(x, table, W):
    ptab = _project(table.T, W)
    idxT = x.T.reshape(B_TOTAL).astype(jnp.int32)
    v2 = _gather(idxT, ptab.reshape(PTAB_ROWS, ROW_W))
    out3 = _unpack(v2.reshape(B_TOTAL // 8, 128))
    return jnp.transpose(out3, (2, 0, 1))


# speedup vs baseline: 3.1877x; 1.1066x over previous
import functools

import jax
import jax.numpy as jnp
from jax import lax
from jax.experimental import pallas as pl
from jax.experimental.pallas import tpu as pltpu
from jax.experimental.pallas import tpu_sc as plsc

NUM_EMB = 1_000_000
D_IN = 64
D_OUT = 32
SCALE = float(D_OUT) ** 0.5

NC = 2
NS = 16
NW = NC * NS

B_TOTAL = 4096 * 200
B_PER_W = B_TOTAL // NW
CHUNK = 128
N_CHUNKS = B_PER_W // CHUNK
K = 8
N_GROUPS = N_CHUNKS // K
GROUP_ROWS = K * CHUNK

ROWS_BLOCK = 32768
N_BLOCKS = pl.cdiv(NUM_EMB, ROWS_BLOCK)
PACK_LINES = ROWS_BLOCK // 8
PTAB_ROWS = N_BLOCKS * ROWS_BLOCK
ROW_W = 16


def _project_body(tabT_ref, w_ref, out_ref):
    p = lax.dot_general(w_ref[...], tabT_ref[...],
                        (((1,), (0,)), ((), ())),
                        preferred_element_type=jnp.float32)
    c = pltpu.bitcast((p * SCALE).astype(jnp.bfloat16), jnp.float32)
    c8 = jnp.concatenate(
        [c[:, s * PACK_LINES:(s + 1) * PACK_LINES] for s in range(8)],
        axis=0)
    out_ref[...] = lax.transpose(c8, (1, 0))


def _project(tabT, w):
    return pl.pallas_call(
        _project_body,
        grid=(N_BLOCKS,),
        in_specs=[
            pl.BlockSpec((D_IN, ROWS_BLOCK), lambda i: (0, i)),
            pl.BlockSpec((D_OUT, D_IN), lambda i: (0, 0)),
        ],
        out_specs=pl.BlockSpec((PACK_LINES, 128), lambda i: (i, 0)),
        out_shape=jax.ShapeDtypeStruct((N_BLOCKS * PACK_LINES, 128), jnp.float32),
    )(tabT, w)


_MESH = plsc.VectorSubcoreMesh(core_axis_name="c", subcore_axis_name="s",
                               num_cores=NC, num_subcores=NS)


N_UNITS = 25
UNIT_ROWS = 1024
STAGE = 8 * 4096


@functools.partial(
    pl.kernel,
    out_type=jax.ShapeDtypeStruct((B_TOTAL, ROW_W), jnp.float32),
    mesh=_MESH,
    compiler_params=pltpu.CompilerParams(use_tc_tiling_on_sc=False,
                                         needs_layout_passes=False),
    scratch_types=[
        pltpu.VMEM((STAGE,), jnp.int32),
        pltpu.VMEM((2 * UNIT_ROWS,), jnp.int32),
        pltpu.VMEM((2 * UNIT_ROWS, ROW_W), jnp.float32),
        pltpu.SemaphoreType.DMA,
        pltpu.SemaphoreType.DMA,
    ],
)
def _gather(idx_hbm, ptab_hbm, out_hbm, idx_v, gidx_v, rows_v, gsem, wsem):
    wid = lax.axis_index("s") * NC + lax.axis_index("c")
    start = jnp.minimum(((wid * N_UNITS) // 4) * 4096, B_TOTAL - STAGE)
    pltpu.sync_copy(idx_hbm.at[pl.ds(start, STAGE)], idx_v)

    lane = lax.iota(jnp.int32, 16)

    def build_and_fire(m, slot):
        mg = wid * N_UNITS + m
        l = mg // 4
        j = mg % 4
        base = l * 4096 + 128 * j - start
        for mv in range(UNIT_ROWS // 16):
            k = 16 * mv + lane
            sp = base + 512 * (k & 7) + (k >> 3)
            v = plsc.load_gather(idx_v, [sp])
            gv = ((v & -32768) + ((v & 4095) << 3)) + ((v >> 12) & 7)
            gidx_v[pl.ds(slot * UNIT_ROWS + 16 * mv, 16)] = gv
        for q in range(8):
            pltpu.async_copy(
                ptab_hbm.at[gidx_v.at[pl.ds(slot * UNIT_ROWS + 128 * q, CHUNK)]],
                rows_v.at[pl.ds(slot * UNIT_ROWS + 128 * q, CHUNK)],
                gsem,
            )

    def drain_gathers(slot):
        for q in range(8):
            pltpu.make_async_copy(
                ptab_hbm.at[gidx_v.at[pl.ds(slot * UNIT_ROWS + 128 * q, CHUNK)]],
                rows_v.at[pl.ds(slot * UNIT_ROWS + 128 * q, CHUNK)],
                gsem,
            ).wait()

    def write_copy(m, slot):
        mg = wid * N_UNITS + m
        dst = 4096 * (mg // 4) + 1024 * (mg % 4)
        return pltpu.make_async_copy(
            rows_v.at[pl.ds(slot * UNIT_ROWS, UNIT_ROWS)],
            out_hbm.at[pl.ds(dst, UNIT_ROWS)],
            wsem,
        )

    build_and_fire(0, 0)

    def unit(m, carry):
        slot = m % 2
        nslot = 1 - slot

        @pl.when(m >= 1)
        def _():
            write_copy(m - 1, nslot).wait()

        @pl.when(m + 1 < N_UNITS)
        def _():
            build_and_fire(m + 1, nslot)

        drain_gathers(slot)
        write_copy(m, slot).start()
        return carry

    lax.fori_loop(0, N_UNITS, unit, 0)
    write_copy(N_UNITS - 1, (N_UNITS - 1) % 2).wait()


UNPACK_L = 8


def _unpack_body(v_ref, out_ref):
    vT = lax.transpose(v_ref[...], (1, 0))
    for li in range(UNPACK_L):
        for u in range(8):
            cs = vT[16 * u:16 * (u + 1), 512 * li:512 * (li + 1)]
            bf = pltpu.bitcast(cs, jnp.bfloat16)
            out_ref[li, :, 512 * u:512 * (u + 1)] = bf.astype(jnp.float32)


def _unpack(v2):
    return pl.pallas_call(
        _unpack_body,
        grid=(200 // UNPACK_L,),
        in_specs=[pl.BlockSpec((512 * UNPACK_L, 128), lambda l: (l, 0))],
        out_specs=pl.BlockSpec((UNPACK_L, D_OUT, 4096), lambda l: (l, 0, 0)),
        out_shape=jax.ShapeDtypeStruct((200, D_OUT, 4096), jnp.float32),
    )(v2)


def kernel(x, table, W):
    ptab = _project(table.T, W)
    idxT = x.T.reshape(B_TOTAL).astype(jnp.int32)
    v2 = _gather(idxT, ptab.reshape(PTAB_ROWS, ROW_W))
    out3 = _unpack(v2.reshape(B_TOTAL // 8, 128))
    return jnp.transpose(out3, (2, 0, 1))


# speedup vs baseline: 3.2014x; 1.0043x over previous
import functools

import jax
import jax.numpy as jnp
from jax import lax
from jax.experimental import pallas as pl
from jax.experimental.pallas import tpu as pltpu
from jax.experimental.pallas import tpu_sc as plsc

NUM_EMB = 1_000_000
D_IN = 64
D_OUT = 32
SCALE = float(D_OUT) ** 0.5

NC = 2
NS = 16
NW = NC * NS

B_TOTAL = 4096 * 200
B_PER_W = B_TOTAL // NW
CHUNK = 128
N_CHUNKS = B_PER_W // CHUNK
K = 8
N_GROUPS = N_CHUNKS // K
GROUP_ROWS = K * CHUNK

ROWS_BLOCK = 65536
N_BLOCKS = pl.cdiv(NUM_EMB, ROWS_BLOCK)
PACK_LINES = ROWS_BLOCK // 8
PTAB_ROWS = N_BLOCKS * ROWS_BLOCK
ROW_W = 16


def _project_body(tabT_ref, w_ref, out_ref):
    p = lax.dot_general(w_ref[...], tabT_ref[...],
                        (((1,), (0,)), ((), ())),
                        preferred_element_type=jnp.float32)
    c = pltpu.bitcast((p * SCALE).astype(jnp.bfloat16), jnp.float32)
    c8 = jnp.concatenate(
        [c[:, s * PACK_LINES:(s + 1) * PACK_LINES] for s in range(8)],
        axis=0)
    out_ref[...] = lax.transpose(c8, (1, 0))


def _project(tabT, w):
    return pl.pallas_call(
        _project_body,
        grid=(N_BLOCKS,),
        in_specs=[
            pl.BlockSpec((D_IN, ROWS_BLOCK), lambda i: (0, i)),
            pl.BlockSpec((D_OUT, D_IN), lambda i: (0, 0)),
        ],
        out_specs=pl.BlockSpec((PACK_LINES, 128), lambda i: (i, 0)),
        out_shape=jax.ShapeDtypeStruct((N_BLOCKS * PACK_LINES, 128), jnp.float32),
    )(tabT, w)


_MESH = plsc.VectorSubcoreMesh(core_axis_name="c", subcore_axis_name="s",
                               num_cores=NC, num_subcores=NS)


N_UNITS = 25
UNIT_ROWS = 1024
STAGE = 8 * 4096


@functools.partial(
    pl.kernel,
    out_type=jax.ShapeDtypeStruct((B_TOTAL, ROW_W), jnp.float32),
    mesh=_MESH,
    compiler_params=pltpu.CompilerParams(use_tc_tiling_on_sc=False,
                                         needs_layout_passes=False),
    scratch_types=[
        pltpu.VMEM((STAGE,), jnp.int32),
        pltpu.VMEM((2 * UNIT_ROWS,), jnp.int32),
        pltpu.VMEM((2 * UNIT_ROWS, ROW_W), jnp.float32),
        pltpu.SemaphoreType.DMA,
        pltpu.SemaphoreType.DMA,
    ],
)
def _gather(idx_hbm, ptab_hbm, out_hbm, idx_v, gidx_v, rows_v, gsem, wsem):
    wid = lax.axis_index("s") * NC + lax.axis_index("c")
    start = jnp.minimum(((wid * N_UNITS) // 4) * 4096, B_TOTAL - STAGE)
    pltpu.sync_copy(idx_hbm.at[pl.ds(start, STAGE)], idx_v)

    lane = lax.iota(jnp.int32, 16)

    def build_and_fire(m, slot):
        mg = wid * N_UNITS + m
        l = mg // 4
        j = mg % 4
        base = l * 4096 + 128 * j - start
        for mv in range(UNIT_ROWS // 16):
            k = 16 * mv + lane
            sp = base + 512 * (k & 7) + (k >> 3)
            v = plsc.load_gather(idx_v, [sp])
            gv = ((v & -65536) + ((v & 8191) << 3)) + ((v >> 13) & 7)
            gidx_v[pl.ds(slot * UNIT_ROWS + 16 * mv, 16)] = gv
        for q in range(8):
            pltpu.async_copy(
                ptab_hbm.at[gidx_v.at[pl.ds(slot * UNIT_ROWS + 128 * q, CHUNK)]],
                rows_v.at[pl.ds(slot * UNIT_ROWS + 128 * q, CHUNK)],
                gsem,
            )

    def drain_gathers(slot):
        for q in range(8):
            pltpu.make_async_copy(
                ptab_hbm.at[gidx_v.at[pl.ds(slot * UNIT_ROWS + 128 * q, CHUNK)]],
                rows_v.at[pl.ds(slot * UNIT_ROWS + 128 * q, CHUNK)],
                gsem,
            ).wait()

    def write_copy(m, slot):
        mg = wid * N_UNITS + m
        dst = 4096 * (mg // 4) + 1024 * (mg % 4)
        return pltpu.make_async_copy(
            rows_v.at[pl.ds(slot * UNIT_ROWS, UNIT_ROWS)],
            out_hbm.at[pl.ds(dst, UNIT_ROWS)],
            wsem,
        )

    build_and_fire(0, 0)

    def unit(m, carry):
        slot = m % 2
        nslot = 1 - slot

        @pl.when(m >= 1)
        def _():
            write_copy(m - 1, nslot).wait()

        @pl.when(m + 1 < N_UNITS)
        def _():
            build_and_fire(m + 1, nslot)

        drain_gathers(slot)
        write_copy(m, slot).start()
        return carry

    lax.fori_loop(0, N_UNITS, unit, 0)
    write_copy(N_UNITS - 1, (N_UNITS - 1) % 2).wait()


UNPACK_L = 10


def _unpack_body(v_ref, out_ref):
    vT = lax.transpose(v_ref[...], (1, 0))
    for li in range(UNPACK_L):
        for u in range(8):
            cs = vT[16 * u:16 * (u + 1), 512 * li:512 * (li + 1)]
            bf = pltpu.bitcast(cs, jnp.bfloat16)
            out_ref[li, :, 512 * u:512 * (u + 1)] = bf.astype(jnp.float32)


def _unpack(v2):
    return pl.pallas_call(
        _unpack_body,
        grid=(200 // UNPACK_L,),
        in_specs=[pl.BlockSpec((512 * UNPACK_L, 128), lambda l: (l, 0))],
        out_specs=pl.BlockSpec((UNPACK_L, D_OUT, 4096), lambda l: (l, 0, 0)),
        out_shape=jax.ShapeDtypeStruct((200, D_OUT, 4096), jnp.float32),
    )(v2)


def kernel(x, table, W):
    ptab = _project(table.T, W)
    idxT = x.T.reshape(B_TOTAL).astype(jnp.int32)
    v2 = _gather(idxT, ptab.reshape(PTAB_ROWS, ROW_W))
    out3 = _unpack(v2.reshape(B_TOTAL // 8, 128))
    return jnp.transpose(out3, (2, 0, 1))


# speedup vs baseline: 3.2943x; 1.0290x over previous
import functools

import jax
import jax.numpy as jnp
from jax import lax
from jax.experimental import pallas as pl
from jax.experimental.pallas import tpu as pltpu
from jax.experimental.pallas import tpu_sc as plsc

NUM_EMB = 1_000_000
D_IN = 64
D_OUT = 32
SCALE = float(D_OUT) ** 0.5

NC = 2
NS = 16
NW = NC * NS

B_TOTAL = 4096 * 200
B_PER_W = B_TOTAL // NW
CHUNK = 128
N_CHUNKS = B_PER_W // CHUNK
K = 8
N_GROUPS = N_CHUNKS // K
GROUP_ROWS = K * CHUNK

ROWS_BLOCK = 65536
N_BLOCKS = pl.cdiv(NUM_EMB, ROWS_BLOCK)
PACK_LINES = ROWS_BLOCK // 8
PTAB_ROWS = N_BLOCKS * ROWS_BLOCK
ROW_W = 16


def _project_body(tabT_ref, w_ref, out_ref):
    p = lax.dot_general(w_ref[...], tabT_ref[...],
                        (((1,), (0,)), ((), ())),
                        preferred_element_type=jnp.float32)
    c = pltpu.bitcast((p * SCALE).astype(jnp.bfloat16), jnp.float32)
    c8 = jnp.concatenate(
        [c[:, s * PACK_LINES:(s + 1) * PACK_LINES] for s in range(8)],
        axis=0)
    out_ref[...] = lax.transpose(c8, (1, 0))


def _project(tabT, w):
    return pl.pallas_call(
        _project_body,
        grid=(N_BLOCKS,),
        in_specs=[
            pl.BlockSpec((D_IN, ROWS_BLOCK), lambda i: (0, i)),
            pl.BlockSpec((D_OUT, D_IN), lambda i: (0, 0)),
        ],
        out_specs=pl.BlockSpec((PACK_LINES, 128), lambda i: (i, 0)),
        out_shape=jax.ShapeDtypeStruct((N_BLOCKS * PACK_LINES, 128), jnp.float32),
    )(tabT, w)


_MESH = plsc.VectorSubcoreMesh(core_axis_name="c", subcore_axis_name="s",
                               num_cores=NC, num_subcores=NS)


N_UNITS = 25
UNIT_ROWS = 1024
STAGE = 8 * 4096


@functools.partial(
    pl.kernel,
    out_type=jax.ShapeDtypeStruct((B_TOTAL,), jnp.int32),
    mesh=_MESH,
    compiler_params=pltpu.CompilerParams(use_tc_tiling_on_sc=False,
                                         needs_layout_passes=False),
    scratch_types=[
        pltpu.VMEM((STAGE,), jnp.int32),
        pltpu.VMEM((B_PER_W,), jnp.int32),
    ],
)
def _prep(idx_hbm, gidx_hbm, idx_v, gout_v):
    wid = lax.axis_index("s") * NC + lax.axis_index("c")
    start = jnp.minimum(((wid * N_UNITS) // 4) * 4096, B_TOTAL - STAGE)
    pltpu.sync_copy(idx_hbm.at[pl.ds(start, STAGE)], idx_v)

    lane = lax.iota(jnp.int32, 16)

    def unit(m, carry):
        mg = wid * N_UNITS + m
        l = mg // 4
        j = mg % 4
        base = l * 4096 + 128 * j - start
        for mv in range(UNIT_ROWS // 16):
            k = 16 * mv + lane
            sp = base + 512 * (k & 7) + (k >> 3)
            v = plsc.load_gather(idx_v, [sp])
            gv = ((v & -65536) + ((v & 8191) << 3)) + ((v >> 13) & 7)
            gout_v[pl.ds(m * UNIT_ROWS + 16 * mv, 16)] = gv
        return carry

    lax.fori_loop(0, N_UNITS, unit, 0)
    pltpu.sync_copy(gout_v, gidx_hbm.at[pl.ds(wid * B_PER_W, B_PER_W)])


@functools.partial(
    pl.kernel,
    out_type=jax.ShapeDtypeStruct((B_TOTAL, ROW_W), jnp.float32),
    mesh=_MESH,
    compiler_params=pltpu.CompilerParams(use_tc_tiling_on_sc=False,
                                         needs_layout_passes=False),
    scratch_types=[
        pltpu.VMEM((B_PER_W,), jnp.int32),
        pltpu.VMEM((2 * UNIT_ROWS, ROW_W), jnp.float32),
        pltpu.SemaphoreType.DMA,
        pltpu.SemaphoreType.DMA,
    ],
)
def _gather(gidx_hbm, ptab_hbm, out_hbm, gidx_v, rows_v, gsem, wsem):
    wid = lax.axis_index("s") * NC + lax.axis_index("c")
    pltpu.sync_copy(gidx_hbm.at[pl.ds(wid * B_PER_W, B_PER_W)], gidx_v)

    def fire_gathers(m, slot):
        for q in range(8):
            pltpu.async_copy(
                ptab_hbm.at[gidx_v.at[pl.ds(m * UNIT_ROWS + 128 * q, CHUNK)]],
                rows_v.at[pl.ds(slot * UNIT_ROWS + 128 * q, CHUNK)],
                gsem,
            )

    def drain_gathers(m, slot):
        for q in range(8):
            pltpu.make_async_copy(
                ptab_hbm.at[gidx_v.at[pl.ds(m * UNIT_ROWS + 128 * q, CHUNK)]],
                rows_v.at[pl.ds(slot * UNIT_ROWS + 128 * q, CHUNK)],
                gsem,
            ).wait()

    def write_copy(m, slot):
        mg = wid * N_UNITS + m
        dst = 4096 * (mg // 4) + 1024 * (mg % 4)
        return pltpu.make_async_copy(
            rows_v.at[pl.ds(slot * UNIT_ROWS, UNIT_ROWS)],
            out_hbm.at[pl.ds(dst, UNIT_ROWS)],
            wsem,
        )

    fire_gathers(0, 0)

    def unit(m, carry):
        slot = m % 2
        nslot = 1 - slot

        @pl.when(m >= 1)
        def _():
            write_copy(m - 1, nslot).wait()

        @pl.when(m + 1 < N_UNITS)
        def _():
            fire_gathers(m + 1, nslot)

        drain_gathers(m, slot)
        write_copy(m, slot).start()
        return carry

    lax.fori_loop(0, N_UNITS, unit, 0)
    write_copy(N_UNITS - 1, (N_UNITS - 1) % 2).wait()


UNPACK_L = 10


def _unpack_body(v_ref, out_ref):
    vT = lax.transpose(v_ref[...], (1, 0))
    for li in range(UNPACK_L):
        for u in range(8):
            cs = vT[16 * u:16 * (u + 1), 512 * li:512 * (li + 1)]
            bf = pltpu.bitcast(cs, jnp.bfloat16)
            out_ref[li, :, 512 * u:512 * (u + 1)] = bf.astype(jnp.float32)


def _unpack(v2):
    return pl.pallas_call(
        _unpack_body,
        grid=(200 // UNPACK_L,),
        in_specs=[pl.BlockSpec((512 * UNPACK_L, 128), lambda l: (l, 0))],
        out_specs=pl.BlockSpec((UNPACK_L, D_OUT, 4096), lambda l: (l, 0, 0)),
        out_shape=jax.ShapeDtypeStruct((200, D_OUT, 4096), jnp.float32),
    )(v2)


def kernel(x, table, W):
    ptab = _project(table.T, W)
    idxT = x.T.reshape(B_TOTAL).astype(jnp.int32)
    gidx = _prep(idxT)
    v2 = _gather(gidx, ptab.reshape(PTAB_ROWS, ROW_W))
    out3 = _unpack(v2.reshape(B_TOTAL // 8, 128))
    return jnp.transpose(out3, (2, 0, 1))
